# Initial kernel scaffold; baseline (speedup 1.0000x reference)
import functools

import jax
import jax.numpy as jnp
from jax import lax
from jax.experimental import pallas as pl
from jax.experimental.pallas import tpu as pltpu
from jax.experimental.pallas import tpu_sc as plsc

N = 10000
E = 320000
D = 128

NC = 2
NS = 16
NW = NC * NS
EPW = E // NW
CH = 80
NCH = EPW // CH
RPT = N // NS
CW = 16

_mesh = plsc.VectorSubcoreMesh(core_axis_name="c", subcore_axis_name="s")


@functools.partial(
    pl.kernel,
    mesh=_mesh,
    out_type=(
        jax.ShapeDtypeStruct((NC, N, D), jnp.float32),
        jax.ShapeDtypeStruct((NC, N, CW), jnp.float32),
    ),
    scratch_types=[
        pltpu.VMEM((CH,), jnp.int32),
        pltpu.VMEM((CH,), jnp.int32),
        pltpu.VMEM((CH, D), jnp.float32),
        pltpu.VMEM((CH, CW), jnp.float32),
        pltpu.VMEM_SHARED((N, D), jnp.float32),
        pltpu.VMEM_SHARED((N, CW), jnp.float32),
        pltpu.SemaphoreType.DMA,
    ],
)
def _agg(x_hbm, src_hbm, dst_hbm, zrow_hbm, zcnt_hbm, ones_hbm,
         acc_out, cnt_out,
         src_v, dst_v, rows_v, ones_v, acc_s, cnt_s, sem):
    cid = lax.axis_index("c")
    sid = lax.axis_index("s")
    wid = sid * NC + cid

    r0 = sid * RPT
    pltpu.sync_copy(zrow_hbm, acc_s.at[pl.ds(r0, RPT)])
    pltpu.sync_copy(zcnt_hbm, cnt_s.at[pl.ds(r0, RPT)])
    pltpu.sync_copy(ones_hbm, ones_v)
    plsc.subcore_barrier()

    ebase = wid * EPW

    def chunk(i, carry):
        off = ebase + i * CH
        pltpu.sync_copy(src_hbm.at[pl.ds(off, CH)], src_v)
        pltpu.sync_copy(dst_hbm.at[pl.ds(off, CH)], dst_v)
        pltpu.async_copy(x_hbm.at[src_v], rows_v, sem).wait()
        pltpu.sync_copy(rows_v, acc_s.at[dst_v], add=True)
        pltpu.sync_copy(ones_v, cnt_s.at[dst_v], add=True)
        return carry

    lax.fori_loop(0, NCH, chunk, 0)
    plsc.subcore_barrier()

    pltpu.sync_copy(acc_s.at[pl.ds(r0, RPT)], acc_out.at[cid, pl.ds(r0, RPT)])
    pltpu.sync_copy(cnt_s.at[pl.ds(r0, RPT)], cnt_out.at[cid, pl.ds(r0, RPT)])


ROWS_BLK = 1000


def _dense_body(x_ref, acc_ref, cnt_ref, wl_ref, wr_ref, b_ref, o_ref):
    acc = acc_ref[0] + acc_ref[1]
    cnt = cnt_ref[0][:, 0:1] + cnt_ref[1][:, 0:1]
    mean = acc / jnp.maximum(cnt, 1.0)
    x = x_ref[...]
    h = (jnp.dot(mean, wl_ref[...], preferred_element_type=jnp.float32)
         + jnp.dot(x, wr_ref[...], preferred_element_type=jnp.float32)
         + b_ref[...])
    o_ref[...] = jnp.maximum(h, 0.0) + x


ROWS_BLK = 1000


def _dense(x, acc_p, cnt_p, Wl, Wr, b):
    grid = (N // ROWS_BLK,)
    return pl.pallas_call(
        _dense_body,
        grid=grid,
        in_specs=[
            pl.BlockSpec((ROWS_BLK, D), lambda i: (i, 0)),
            pl.BlockSpec((NC, ROWS_BLK, D), lambda i: (0, i, 0)),
            pl.BlockSpec((NC, ROWS_BLK, CW), lambda i: (0, i, 0)),
            pl.BlockSpec((D, D), lambda i: (0, 0)),
            pl.BlockSpec((D, D), lambda i: (0, 0)),
            pl.BlockSpec((1, D), lambda i: (0, 0)),
        ],
        out_specs=pl.BlockSpec((ROWS_BLK, D), lambda i: (i, 0)),
        out_shape=jax.ShapeDtypeStruct((N, D), jnp.float32),
    )(x, acc_p, cnt_p, Wl, Wr, b.reshape(1, D))


def kernel(x, edge_index, Wl0, Wr0, b0, Wl1, Wr1, b1):
    src = edge_index[0]
    dst = edge_index[1]
    zrow = jnp.zeros((RPT, D), jnp.float32)
    zcnt = jnp.zeros((RPT, CW), jnp.float32)
    ones = jnp.ones((CH, CW), jnp.float32)

    acc_p, cnt_p = _agg(x, src, dst, zrow, zcnt, ones)
    h0 = _dense(x, acc_p, cnt_p, Wl0, Wr0, b0)
    acc_p1, cnt_p1 = _agg(h0, src, dst, zrow, zcnt, ones)
    out = _dense(h0, acc_p1, cnt_p1, Wl1, Wr1, b1)
    return (out, edge_index)


# speedup vs baseline: 4.7325x; 4.7325x over previous
import functools

import jax
import jax.numpy as jnp
from jax import lax
from jax.experimental import pallas as pl
from jax.experimental.pallas import tpu as pltpu
from jax.experimental.pallas import tpu_sc as plsc

N = 10000
E = 320000
D = 128

NC = 2
NS = 16
NW = NC * NS
EPW = E // NW
CH = 80
NCH = EPW // CH
N_PAD = 10240
RPT = N_PAD // NS
NRC = RPT // CH
CR = N_PAD // D

_mesh = plsc.VectorSubcoreMesh(core_axis_name="c", subcore_axis_name="s")


@functools.partial(
    pl.kernel,
    mesh=_mesh,
    out_type=jax.ShapeDtypeStruct((NC, N_PAD, D), jnp.float32),
    scratch_types=[
        pltpu.VMEM((CH,), jnp.int32),
        pltpu.VMEM((CH,), jnp.int32),
        pltpu.VMEM((CH,), jnp.int32),
        pltpu.VMEM((CH, D), jnp.float32),
        pltpu.VMEM_SHARED((N_PAD, D), jnp.float32),
        pltpu.SemaphoreType.DMA,
    ],
)
def _agg(x_hbm, src_hbm, dst_hbm, zrow_hbm,
         acc_out,
         src_v, dst_v, idx_v, rows_v, acc_s, sem):
    cid = lax.axis_index("c")
    sid = lax.axis_index("s")
    wid = sid * NC + cid
    r0 = sid * RPT

    def fill_rowids(base):
        for k in range(CH // 16):
            idx_v[pl.ds(k * 16, 16)] = base + k * 16 + lax.iota(jnp.int32, 16)

    pltpu.sync_copy(zrow_hbm, rows_v)
    for j in range(NRC):
        fill_rowids(r0 + j * CH)
        pltpu.sync_copy(rows_v, acc_s.at[idx_v])
    plsc.subcore_barrier()

    ebase = wid * EPW

    def chunk(i, carry):
        off = ebase + i * CH
        pltpu.sync_copy(src_hbm.at[pl.ds(off, CH)], src_v)
        pltpu.sync_copy(dst_hbm.at[pl.ds(off, CH)], dst_v)
        pltpu.async_copy(x_hbm.at[src_v], rows_v, sem).wait()
        pltpu.sync_copy(rows_v, acc_s.at[dst_v], add=True)
        return carry

    lax.fori_loop(0, NCH, chunk, 0)
    plsc.subcore_barrier()

    for j in range(NRC):
        fill_rowids(r0 + j * CH)
        pltpu.async_copy(acc_s.at[idx_v], rows_v, sem).wait()
        pltpu.sync_copy(rows_v, acc_out.at[cid, pl.ds(r0 + j * CH, CH)])


@functools.partial(
    pl.kernel,
    mesh=_mesh,
    out_type=jax.ShapeDtypeStruct((NC, N_PAD, D), jnp.float32),
    scratch_types=[
        pltpu.VMEM((CH,), jnp.int32),
        pltpu.VMEM((CH,), jnp.int32),
        pltpu.VMEM((CH, D), jnp.float32),
        pltpu.VMEM((CH, D), jnp.float32),
        pltpu.VMEM_SHARED((N_PAD, D), jnp.float32),
        pltpu.SemaphoreType.DMA,
    ],
)
def _cnt(dst_hbm, zrow_hbm, ones_hbm,
         cnt_out,
         dst_v, idx_v, ones_v, zbuf_v, cnt_s, sem):
    cid = lax.axis_index("c")
    sid = lax.axis_index("s")
    wid = sid * NC + cid
    r0 = sid * RPT

    def fill_rowids(base):
        for k in range(CH // 16):
            idx_v[pl.ds(k * 16, 16)] = base + k * 16 + lax.iota(jnp.int32, 16)

    pltpu.sync_copy(zrow_hbm, zbuf_v)
    pltpu.sync_copy(ones_hbm, ones_v)
    for j in range(NRC):
        fill_rowids(r0 + j * CH)
        pltpu.sync_copy(zbuf_v, cnt_s.at[idx_v])
    plsc.subcore_barrier()

    ebase = wid * EPW

    def chunk(i, carry):
        off = ebase + i * CH
        pltpu.sync_copy(dst_hbm.at[pl.ds(off, CH)], dst_v)
        pltpu.sync_copy(ones_v, cnt_s.at[dst_v], add=True)
        return carry

    lax.fori_loop(0, NCH, chunk, 0)
    plsc.subcore_barrier()

    for j in range(NRC):
        fill_rowids(r0 + j * CH)
        pltpu.async_copy(cnt_s.at[idx_v], zbuf_v, sem).wait()
        pltpu.sync_copy(zbuf_v, cnt_out.at[cid, pl.ds(r0 + j * CH, CH)])


ROWS_BLK = 1000


def _dense_body(x_ref, acc_ref, cnt_ref, wl_ref, wr_ref, b_ref, o_ref):
    acc = acc_ref[0] + acc_ref[1]
    cnt = cnt_ref[0] + cnt_ref[1]
    mean = acc / jnp.maximum(cnt, 1.0)
    x = x_ref[...]
    h = (jnp.dot(mean, wl_ref[...], preferred_element_type=jnp.float32)
         + jnp.dot(x, wr_ref[...], preferred_element_type=jnp.float32)
         + b_ref[...])
    o_ref[...] = jnp.maximum(h, 0.0) + x


def _dense(x, acc_p, cnt_p, Wl, Wr, b):
    grid = (N // ROWS_BLK,)
    return pl.pallas_call(
        _dense_body,
        grid=grid,
        in_specs=[
            pl.BlockSpec((ROWS_BLK, D), lambda i: (i, 0)),
            pl.BlockSpec((NC, ROWS_BLK, D), lambda i: (0, i, 0)),
            pl.BlockSpec((NC, ROWS_BLK, 1), lambda i: (0, i, 0)),
            pl.BlockSpec((D, D), lambda i: (0, 0)),
            pl.BlockSpec((D, D), lambda i: (0, 0)),
            pl.BlockSpec((1, D), lambda i: (0, 0)),
        ],
        out_specs=pl.BlockSpec((ROWS_BLK, D), lambda i: (i, 0)),
        out_shape=jax.ShapeDtypeStruct((N, D), jnp.float32),
    )(x, acc_p, cnt_p, Wl, Wr, b.reshape(1, D))


def kernel(x, edge_index, Wl0, Wr0, b0, Wl1, Wr1, b1):
    src = edge_index[0]
    dst = edge_index[1]
    zrow = jnp.zeros((CH, D), jnp.float32)
    ones = jnp.ones((CH, D), jnp.float32)

    acc_p = _agg(x, src, dst, zrow)
    cnt_p = _cnt(dst, zrow, ones)
    cnt_col = cnt_p[:, :, :1]
    h0 = _dense(x, acc_p, cnt_col, Wl0, Wr0, b0)
    acc_p1 = _agg(h0, src, dst, zrow)
    out = _dense(h0, acc_p1, cnt_col, Wl1, Wr1, b1)
    return (out, edge_index)


# speedup vs baseline: 5.2767x; 1.1150x over previous
import functools

import jax
import jax.numpy as jnp
from jax import lax
from jax.experimental import pallas as pl
from jax.experimental.pallas import tpu as pltpu
from jax.experimental.pallas import tpu_sc as plsc

N = 10000
E = 320000
D = 128

NC = 2
NS = 16
NW = NC * NS
EPW = E // NW
CH = 80
CE = 80
NCE = EPW // CE
N_PAD = 10240
RPT = N_PAD // NS
NRC = RPT // CH
CR = N_PAD // D

_mesh = plsc.VectorSubcoreMesh(core_axis_name="c", subcore_axis_name="s")


@functools.partial(
    pl.kernel,
    mesh=_mesh,
    out_type=jax.ShapeDtypeStruct((NC, N_PAD, D), jnp.float32),
    scratch_types=[
        pltpu.VMEM((CE,), jnp.int32),
        pltpu.VMEM((CE,), jnp.int32),
        pltpu.VMEM((CE,), jnp.int32),
        pltpu.VMEM((CE,), jnp.int32),
        pltpu.VMEM((CH,), jnp.int32),
        pltpu.VMEM((CE, D), jnp.float32),
        pltpu.VMEM((CE, D), jnp.float32),
        pltpu.VMEM((CH, D), jnp.float32),
        pltpu.VMEM_SHARED((N_PAD, D), jnp.float32),
        pltpu.SemaphoreType.DMA,
        pltpu.SemaphoreType.DMA,
    ],
)
def _agg(x_hbm, src_hbm, dst_hbm, zrow_hbm,
         acc_out,
         src_va, dst_va, src_vb, dst_vb,
         idx_v, rows_a, rows_b, bounce_v, acc_s, sem_a, sem_b):
    cid = lax.axis_index("c")
    sid = lax.axis_index("s")
    wid = sid * NC + cid
    r0 = sid * RPT

    def fill_rowids(base):
        for k in range(CH // 16):
            idx_v[pl.ds(k * 16, 16)] = base + k * 16 + lax.iota(jnp.int32, 16)

    ebase = wid * EPW

    pltpu.sync_copy(zrow_hbm, bounce_v)
    for j in range(NRC):
        fill_rowids(r0 + j * CH)
        pltpu.sync_copy(bounce_v, acc_s.at[idx_v])
    plsc.subcore_barrier()

    def pair(j, carry):
        offa = ebase + (2 * j) * CE
        offb = offa + CE
        pltpu.sync_copy(src_hbm.at[pl.ds(offa, CE)], src_va)
        pltpu.sync_copy(dst_hbm.at[pl.ds(offa, CE)], dst_va)
        pltpu.sync_copy(src_hbm.at[pl.ds(offb, CE)], src_vb)
        pltpu.sync_copy(dst_hbm.at[pl.ds(offb, CE)], dst_vb)
        ha = pltpu.async_copy(x_hbm.at[src_va], rows_a, sem_a)
        hb = pltpu.async_copy(x_hbm.at[src_vb], rows_b, sem_b)
        ha.wait()
        pltpu.sync_copy(rows_a, acc_s.at[dst_va], add=True)
        hb.wait()
        pltpu.sync_copy(rows_b, acc_s.at[dst_vb], add=True)
        return carry

    lax.fori_loop(0, NCE // 2, pair, 0)

    offt = ebase + (NCE - 1) * CE
    pltpu.sync_copy(src_hbm.at[pl.ds(offt, CE)], src_va)
    pltpu.sync_copy(dst_hbm.at[pl.ds(offt, CE)], dst_va)
    pltpu.async_copy(x_hbm.at[src_va], rows_a, sem_a).wait()
    pltpu.sync_copy(rows_a, acc_s.at[dst_va], add=True)
    plsc.subcore_barrier()

    for j in range(NRC):
        fill_rowids(r0 + j * CH)
        pltpu.async_copy(acc_s.at[idx_v], bounce_v, sem_a).wait()
        pltpu.sync_copy(bounce_v, acc_out.at[cid, pl.ds(r0 + j * CH, CH)])


@functools.partial(
    pl.kernel,
    mesh=_mesh,
    out_type=jax.ShapeDtypeStruct((NC, N_PAD, D), jnp.float32),
    scratch_types=[
        pltpu.VMEM((CH,), jnp.int32),
        pltpu.VMEM((CH,), jnp.int32),
        pltpu.VMEM((CH, D), jnp.float32),
        pltpu.VMEM((CH, D), jnp.float32),
        pltpu.VMEM_SHARED((N_PAD, D), jnp.float32),
        pltpu.SemaphoreType.DMA,
    ],
)
def _cnt(dst_hbm, zrow_hbm, ones_hbm,
         cnt_out,
         dst_v, idx_v, ones_v, zbuf_v, cnt_s, sem):
    cid = lax.axis_index("c")
    sid = lax.axis_index("s")
    wid = sid * NC + cid
    r0 = sid * RPT

    def fill_rowids(base):
        for k in range(CH // 16):
            idx_v[pl.ds(k * 16, 16)] = base + k * 16 + lax.iota(jnp.int32, 16)

    pltpu.sync_copy(zrow_hbm, zbuf_v)
    pltpu.sync_copy(ones_hbm, ones_v)
    for j in range(NRC):
        fill_rowids(r0 + j * CH)
        pltpu.sync_copy(zbuf_v, cnt_s.at[idx_v])
    plsc.subcore_barrier()

    ebase = wid * EPW

    def chunk(i, carry):
        off = ebase + i * CH
        pltpu.sync_copy(dst_hbm.at[pl.ds(off, CH)], dst_v)
        pltpu.sync_copy(ones_v, cnt_s.at[dst_v], add=True)
        return carry

    lax.fori_loop(0, EPW // CH, chunk, 0)
    plsc.subcore_barrier()

    for j in range(NRC):
        fill_rowids(r0 + j * CH)
        pltpu.async_copy(cnt_s.at[idx_v], zbuf_v, sem).wait()
        pltpu.sync_copy(zbuf_v, cnt_out.at[cid, pl.ds(r0 + j * CH, CH)])


ROWS_BLK = 1000


def _dense_body(x_ref, acc_ref, cnt_ref, wl_ref, wr_ref, b_ref, o_ref):
    acc = acc_ref[0] + acc_ref[1]
    cnt = cnt_ref[0] + cnt_ref[1]
    mean = acc / jnp.maximum(cnt, 1.0)
    x = x_ref[...]
    h = (jnp.dot(mean, wl_ref[...], preferred_element_type=jnp.float32)
         + jnp.dot(x, wr_ref[...], preferred_element_type=jnp.float32)
         + b_ref[...])
    o_ref[...] = jnp.maximum(h, 0.0) + x


def _dense(x, acc_p, cnt_p, Wl, Wr, b):
    grid = (N // ROWS_BLK,)
    return pl.pallas_call(
        _dense_body,
        grid=grid,
        in_specs=[
            pl.BlockSpec((ROWS_BLK, D), lambda i: (i, 0)),
            pl.BlockSpec((NC, ROWS_BLK, D), lambda i: (0, i, 0)),
            pl.BlockSpec((NC, ROWS_BLK, 1), lambda i: (0, i, 0)),
            pl.BlockSpec((D, D), lambda i: (0, 0)),
            pl.BlockSpec((D, D), lambda i: (0, 0)),
            pl.BlockSpec((1, D), lambda i: (0, 0)),
        ],
        out_specs=pl.BlockSpec((ROWS_BLK, D), lambda i: (i, 0)),
        out_shape=jax.ShapeDtypeStruct((N, D), jnp.float32),
    )(x, acc_p, cnt_p, Wl, Wr, b.reshape(1, D))


def kernel(x, edge_index, Wl0, Wr0, b0, Wl1, Wr1, b1):
    src = edge_index[0]
    dst = edge_index[1]
    zrow = jnp.zeros((CH, D), jnp.float32)
    ones = jnp.ones((CH, D), jnp.float32)

    acc_p = _agg(x, src, dst, zrow)
    cnt_p = _cnt(dst, zrow, ones)
    cnt_col = cnt_p[:, :, :1]
    h0 = _dense(x, acc_p, cnt_col, Wl0, Wr0, b0)
    acc_p1 = _agg(h0, src, dst, zrow)
    out = _dense(h0, acc_p1, cnt_col, Wl1, Wr1, b1)
    return (out, edge_index)


# speedup vs baseline: 6.3392x; 1.2013x over previous
import functools

import jax
import jax.numpy as jnp
from jax import lax
from jax.experimental import pallas as pl
from jax.experimental.pallas import tpu as pltpu
from jax.experimental.pallas import tpu_sc as plsc

N = 10000
E = 320000
D = 128

NC = 2
NS = 16
NW = NC * NS
EPW = E // NW
CH = 80
CE = 128
NCE = EPW // CE
TL = EPW - NCE * CE
N_PAD = 10240
RPT = N_PAD // NS
NRC = RPT // CH

_mesh = plsc.VectorSubcoreMesh(core_axis_name="c", subcore_axis_name="s")


@functools.partial(
    pl.kernel,
    mesh=_mesh,
    out_type=jax.ShapeDtypeStruct((NC, N_PAD, D), jnp.float32),
    scratch_types=[
        pltpu.VMEM((CE,), jnp.int32),
        pltpu.VMEM((CE,), jnp.int32),
        pltpu.VMEM((CE,), jnp.int32),
        pltpu.VMEM((CE,), jnp.int32),
        pltpu.VMEM((TL,), jnp.int32),
        pltpu.VMEM((TL,), jnp.int32),
        pltpu.VMEM((CH,), jnp.int32),
        pltpu.VMEM((CE, D), jnp.float32),
        pltpu.VMEM((CE, D), jnp.float32),
        pltpu.VMEM((TL, D), jnp.float32),
        pltpu.VMEM((CH, D), jnp.float32),
        pltpu.VMEM_SHARED((N_PAD, D), jnp.float32),
        pltpu.SemaphoreType.DMA,
        pltpu.SemaphoreType.DMA,
    ],
)
def _agg(x_hbm, src_hbm, dst_hbm, zrow_hbm,
         acc_out,
         src_va, dst_va, src_vb, dst_vb, src_vt, dst_vt,
         idx_v, rows_a, rows_b, rows_t, bounce_v, acc_s, sem_a, sem_b):
    cid = lax.axis_index("c")
    sid = lax.axis_index("s")
    wid = sid * NC + cid
    r0 = sid * RPT

    def fill_rowids(base):
        for k in range(CH // 16):
            idx_v[pl.ds(k * 16, 16)] = base + k * 16 + lax.iota(jnp.int32, 16)

    ebase = wid * EPW

    pltpu.sync_copy(zrow_hbm, bounce_v)
    for j in range(NRC):
        fill_rowids(r0 + j * CH)
        pltpu.sync_copy(bounce_v, acc_s.at[idx_v])
    plsc.subcore_barrier()

    def pair(j, carry):
        offa = ebase + (2 * j) * CE
        offb = offa + CE
        pltpu.sync_copy(src_hbm.at[pl.ds(offa, CE)], src_va)
        pltpu.sync_copy(dst_hbm.at[pl.ds(offa, CE)], dst_va)
        pltpu.sync_copy(src_hbm.at[pl.ds(offb, CE)], src_vb)
        pltpu.sync_copy(dst_hbm.at[pl.ds(offb, CE)], dst_vb)
        ha = pltpu.async_copy(x_hbm.at[src_va], rows_a, sem_a)
        hb = pltpu.async_copy(x_hbm.at[src_vb], rows_b, sem_b)
        ha.wait()
        pltpu.sync_copy(rows_a, acc_s.at[dst_va], add=True)
        hb.wait()
        pltpu.sync_copy(rows_b, acc_s.at[dst_vb], add=True)
        return carry

    lax.fori_loop(0, NCE // 2, pair, 0)

    offt = ebase + NCE * CE
    pltpu.sync_copy(src_hbm.at[pl.ds(offt, TL)], src_vt)
    pltpu.sync_copy(dst_hbm.at[pl.ds(offt, TL)], dst_vt)
    pltpu.async_copy(x_hbm.at[src_vt], rows_t, sem_a).wait()
    pltpu.sync_copy(rows_t, acc_s.at[dst_vt], add=True)
    plsc.subcore_barrier()

    for j in range(NRC):
        fill_rowids(r0 + j * CH)
        pltpu.async_copy(acc_s.at[idx_v], bounce_v, sem_a).wait()
        pltpu.sync_copy(bounce_v, acc_out.at[cid, pl.ds(r0 + j * CH, CH)])


@functools.partial(
    pl.kernel,
    mesh=_mesh,
    out_type=jax.ShapeDtypeStruct((NC, N_PAD, D), jnp.float32),
    scratch_types=[
        pltpu.VMEM((CE,), jnp.int32),
        pltpu.VMEM((CE,), jnp.int32),
        pltpu.VMEM((TL,), jnp.int32),
        pltpu.VMEM((CH,), jnp.int32),
        pltpu.VMEM((CE, D), jnp.float32),
        pltpu.VMEM((TL, D), jnp.float32),
        pltpu.VMEM((CH, D), jnp.float32),
        pltpu.VMEM_SHARED((N_PAD, D), jnp.float32),
        pltpu.SemaphoreType.DMA,
        pltpu.SemaphoreType.DMA,
    ],
)
def _cnt(dst_hbm, zrow_hbm, ones_hbm,
         cnt_out,
         dst_va, dst_vb, dst_vt, idx_v, ones_v, ones_t, zbuf_v, cnt_s,
         sem_a, sem_b):
    cid = lax.axis_index("c")
    sid = lax.axis_index("s")
    wid = sid * NC + cid
    r0 = sid * RPT

    def fill_rowids(base):
        for k in range(CH // 16):
            idx_v[pl.ds(k * 16, 16)] = base + k * 16 + lax.iota(jnp.int32, 16)

    pltpu.sync_copy(zrow_hbm, zbuf_v)
    pltpu.sync_copy(ones_hbm, ones_v)
    pltpu.sync_copy(ones_hbm.at[pl.ds(0, TL)], ones_t)
    for j in range(NRC):
        fill_rowids(r0 + j * CH)
        pltpu.sync_copy(zbuf_v, cnt_s.at[idx_v])
    plsc.subcore_barrier()

    ebase = wid * EPW

    def pair(j, carry):
        offa = ebase + (2 * j) * CE
        offb = offa + CE
        ha = pltpu.async_copy(dst_hbm.at[pl.ds(offa, CE)], dst_va, sem_a)
        hb = pltpu.async_copy(dst_hbm.at[pl.ds(offb, CE)], dst_vb, sem_b)
        ha.wait()
        pltpu.sync_copy(ones_v, cnt_s.at[dst_va], add=True)
        hb.wait()
        pltpu.sync_copy(ones_v, cnt_s.at[dst_vb], add=True)
        return carry

    lax.fori_loop(0, NCE // 2, pair, 0)

    offt = ebase + NCE * CE
    pltpu.sync_copy(dst_hbm.at[pl.ds(offt, TL)], dst_vt)
    pltpu.sync_copy(ones_t, cnt_s.at[dst_vt], add=True)
    plsc.subcore_barrier()

    for j in range(NRC):
        fill_rowids(r0 + j * CH)
        pltpu.async_copy(cnt_s.at[idx_v], zbuf_v, sem_a).wait()
        pltpu.sync_copy(zbuf_v, cnt_out.at[cid, pl.ds(r0 + j * CH, CH)])


ROWS_BLK = 1000


def _dense_body(x_ref, acc_ref, cnt_ref, wl_ref, wr_ref, b_ref, o_ref):
    acc = acc_ref[0] + acc_ref[1]
    cnt = cnt_ref[0] + cnt_ref[1]
    mean = acc / jnp.maximum(cnt, 1.0)
    x = x_ref[...]
    h = (jnp.dot(mean, wl_ref[...], preferred_element_type=jnp.float32)
         + jnp.dot(x, wr_ref[...], preferred_element_type=jnp.float32)
         + b_ref[...])
    o_ref[...] = jnp.maximum(h, 0.0) + x


def _dense(x, acc_p, cnt_p, Wl, Wr, b):
    grid = (N // ROWS_BLK,)
    return pl.pallas_call(
        _dense_body,
        grid=grid,
        in_specs=[
            pl.BlockSpec((ROWS_BLK, D), lambda i: (i, 0)),
            pl.BlockSpec((NC, ROWS_BLK, D), lambda i: (0, i, 0)),
            pl.BlockSpec((NC, ROWS_BLK, 1), lambda i: (0, i, 0)),
            pl.BlockSpec((D, D), lambda i: (0, 0)),
            pl.BlockSpec((D, D), lambda i: (0, 0)),
            pl.BlockSpec((1, D), lambda i: (0, 0)),
        ],
        out_specs=pl.BlockSpec((ROWS_BLK, D), lambda i: (i, 0)),
        out_shape=jax.ShapeDtypeStruct((N, D), jnp.float32),
    )(x, acc_p, cnt_p, Wl, Wr, b.reshape(1, D))


def kernel(x, edge_index, Wl0, Wr0, b0, Wl1, Wr1, b1):
    src = edge_index[0]
    dst = edge_index[1]
    zrow = jnp.zeros((CH, D), jnp.float32)
    ones = jnp.ones((CE, D), jnp.float32)

    acc_p = _agg(x, src, dst, zrow)
    cnt_p = _cnt(dst, zrow, ones)
    cnt_col = cnt_p[:, :, :1]
    h0 = _dense(x, acc_p, cnt_col, Wl0, Wr0, b0)
    acc_p1 = _agg(h0, src, dst, zrow)
    out = _dense(h0, acc_p1, cnt_col, Wl1, Wr1, b1)
    return (out, edge_index)


# speedup vs baseline: 7.7736x; 1.2263x over previous
import functools

import jax
import jax.numpy as jnp
from jax import lax
from jax.experimental import pallas as pl
from jax.experimental.pallas import tpu as pltpu
from jax.experimental.pallas import tpu_sc as plsc

N = 10000
E = 320000
D = 128

NC = 2
NS = 16
NW = NC * NS
EPW = E // NW
CH = 80
CE = 128
NCE = EPW // CE
TL = EPW - NCE * CE
N_PAD = 10240
RPT = N_PAD // NS
NRC = RPT // CH

_mesh = plsc.VectorSubcoreMesh(core_axis_name="c", subcore_axis_name="s")


@functools.partial(
    pl.kernel,
    mesh=_mesh,
    out_type=jax.ShapeDtypeStruct((NC, N_PAD, D), jnp.float32),
    scratch_types=[
        pltpu.VMEM((CE,), jnp.int32),
        pltpu.VMEM((CE,), jnp.int32),
        pltpu.VMEM((CE,), jnp.int32),
        pltpu.VMEM((CE,), jnp.int32),
        pltpu.VMEM((TL,), jnp.int32),
        pltpu.VMEM((TL,), jnp.int32),
        pltpu.VMEM((CH,), jnp.int32),
        pltpu.VMEM((CE, D), jnp.float32),
        pltpu.VMEM((CE, D), jnp.float32),
        pltpu.VMEM((TL, D), jnp.float32),
        pltpu.VMEM((CH, D), jnp.float32),
        pltpu.VMEM_SHARED((N_PAD, D), jnp.float32),
        pltpu.SemaphoreType.DMA,
        pltpu.SemaphoreType.DMA,
        pltpu.SemaphoreType.DMA,
    ],
)
def _agg(x_hbm, src_hbm, dst_hbm, zrow_hbm,
         acc_out,
         src_va, dst_va, src_vb, dst_vb, src_vt, dst_vt,
         idx_v, rows_a, rows_b, rows_t, bounce_v, acc_s, sem_a, sem_b, sem_i):
    cid = lax.axis_index("c")
    sid = lax.axis_index("s")
    wid = sid * NC + cid
    r0 = sid * RPT

    def fill_rowids(base):
        for k in range(CH // 16):
            idx_v[pl.ds(k * 16, 16)] = base + k * 16 + lax.iota(jnp.int32, 16)

    ebase = wid * EPW

    pltpu.sync_copy(zrow_hbm, bounce_v)
    for j in range(NRC):
        fill_rowids(r0 + j * CH)
        pltpu.sync_copy(bounce_v, acc_s.at[idx_v])
    plsc.subcore_barrier()

    def pair(j, carry):
        offa = ebase + (2 * j) * CE
        offb = offa + CE
        i1 = pltpu.async_copy(src_hbm.at[pl.ds(offa, CE)], src_va, sem_i)
        i2 = pltpu.async_copy(dst_hbm.at[pl.ds(offa, CE)], dst_va, sem_i)
        i3 = pltpu.async_copy(src_hbm.at[pl.ds(offb, CE)], src_vb, sem_i)
        i4 = pltpu.async_copy(dst_hbm.at[pl.ds(offb, CE)], dst_vb, sem_i)
        i1.wait()
        i2.wait()
        i3.wait()
        i4.wait()
        ha = pltpu.async_copy(x_hbm.at[src_va], rows_a, sem_a)
        hb = pltpu.async_copy(x_hbm.at[src_vb], rows_b, sem_b)
        ha.wait()
        pltpu.sync_copy(rows_a, acc_s.at[dst_va], add=True)
        hb.wait()
        pltpu.sync_copy(rows_b, acc_s.at[dst_vb], add=True)
        return carry

    lax.fori_loop(0, NCE // 2, pair, 0)

    offt = ebase + NCE * CE
    pltpu.sync_copy(src_hbm.at[pl.ds(offt, TL)], src_vt)
    pltpu.sync_copy(dst_hbm.at[pl.ds(offt, TL)], dst_vt)
    pltpu.async_copy(x_hbm.at[src_vt], rows_t, sem_a).wait()
    pltpu.sync_copy(rows_t, acc_s.at[dst_vt], add=True)
    plsc.subcore_barrier()

    for j in range(NRC):
        fill_rowids(r0 + j * CH)
        pltpu.async_copy(acc_s.at[idx_v], bounce_v, sem_a).wait()
        pltpu.sync_copy(bounce_v, acc_out.at[cid, pl.ds(r0 + j * CH, CH)])


@functools.partial(
    pl.kernel,
    mesh=_mesh,
    out_type=jax.ShapeDtypeStruct((NC, N_PAD, D), jnp.float32),
    scratch_types=[
        pltpu.VMEM((CE,), jnp.int32),
        pltpu.VMEM((CE,), jnp.int32),
        pltpu.VMEM((TL,), jnp.int32),
        pltpu.VMEM((CH,), jnp.int32),
        pltpu.VMEM((CE, D), jnp.float32),
        pltpu.VMEM((TL, D), jnp.float32),
        pltpu.VMEM((CH, D), jnp.float32),
        pltpu.VMEM_SHARED((N_PAD, D), jnp.float32),
        pltpu.SemaphoreType.DMA,
        pltpu.SemaphoreType.DMA,
    ],
)
def _cnt(dst_hbm, zrow_hbm, ones_hbm,
         cnt_out,
         dst_va, dst_vb, dst_vt, idx_v, ones_v, ones_t, zbuf_v, cnt_s,
         sem_a, sem_b):
    cid = lax.axis_index("c")
    sid = lax.axis_index("s")
    wid = sid * NC + cid
    r0 = sid * RPT

    def fill_rowids(base):
        for k in range(CH // 16):
            idx_v[pl.ds(k * 16, 16)] = base + k * 16 + lax.iota(jnp.int32, 16)

    pltpu.sync_copy(zrow_hbm, zbuf_v)
    pltpu.sync_copy(ones_hbm, ones_v)
    pltpu.sync_copy(ones_hbm.at[pl.ds(0, TL)], ones_t)
    for j in range(NRC):
        fill_rowids(r0 + j * CH)
        pltpu.sync_copy(zbuf_v, cnt_s.at[idx_v])
    plsc.subcore_barrier()

    ebase = wid * EPW

    def pair(j, carry):
        offa = ebase + (2 * j) * CE
        offb = offa + CE
        ha = pltpu.async_copy(dst_hbm.at[pl.ds(offa, CE)], dst_va, sem_a)
        hb = pltpu.async_copy(dst_hbm.at[pl.ds(offb, CE)], dst_vb, sem_b)
        ha.wait()
        pltpu.sync_copy(ones_v, cnt_s.at[dst_va], add=True)
        hb.wait()
        pltpu.sync_copy(ones_v, cnt_s.at[dst_vb], add=True)
        return carry

    lax.fori_loop(0, NCE // 2, pair, 0)

    offt = ebase + NCE * CE
    pltpu.sync_copy(dst_hbm.at[pl.ds(offt, TL)], dst_vt)
    pltpu.sync_copy(ones_t, cnt_s.at[dst_vt], add=True)
    plsc.subcore_barrier()

    for j in range(NRC):
        fill_rowids(r0 + j * CH)
        pltpu.async_copy(cnt_s.at[idx_v], zbuf_v, sem_a).wait()
        pltpu.sync_copy(zbuf_v, cnt_out.at[cid, pl.ds(r0 + j * CH, CH)])


ROWS_BLK = 1000


def _dense_body(x_ref, acc_ref, cnt_ref, wl_ref, wr_ref, b_ref, o_ref):
    acc = acc_ref[0] + acc_ref[1]
    cnt = cnt_ref[0] + cnt_ref[1]
    mean = acc / jnp.maximum(cnt, 1.0)
    x = x_ref[...]
    h = (jnp.dot(mean, wl_ref[...], preferred_element_type=jnp.float32)
         + jnp.dot(x, wr_ref[...], preferred_element_type=jnp.float32)
         + b_ref[...])
    o_ref[...] = jnp.maximum(h, 0.0) + x


def _dense(x, acc_p, cnt_p, Wl, Wr, b):
    grid = (N // ROWS_BLK,)
    return pl.pallas_call(
        _dense_body,
        grid=grid,
        in_specs=[
            pl.BlockSpec((ROWS_BLK, D), lambda i: (i, 0)),
            pl.BlockSpec((NC, ROWS_BLK, D), lambda i: (0, i, 0)),
            pl.BlockSpec((NC, ROWS_BLK, 1), lambda i: (0, i, 0)),
            pl.BlockSpec((D, D), lambda i: (0, 0)),
            pl.BlockSpec((D, D), lambda i: (0, 0)),
            pl.BlockSpec((1, D), lambda i: (0, 0)),
        ],
        out_specs=pl.BlockSpec((ROWS_BLK, D), lambda i: (i, 0)),
        out_shape=jax.ShapeDtypeStruct((N, D), jnp.float32),
    )(x, acc_p, cnt_p, Wl, Wr, b.reshape(1, D))


def kernel(x, edge_index, Wl0, Wr0, b0, Wl1, Wr1, b1):
    src = edge_index[0]
    dst = edge_index[1]
    zrow = jnp.zeros((CH, D), jnp.float32)
    ones = jnp.ones((CE, D), jnp.float32)

    acc_p = _agg(x, src, dst, zrow)
    cnt_p = _cnt(dst, zrow, ones)
    cnt_col = cnt_p[:, :, :1]
    h0 = _dense(x, acc_p, cnt_col, Wl0, Wr0, b0)
    acc_p1 = _agg(h0, src, dst, zrow)
    out = _dense(h0, acc_p1, cnt_col, Wl1, Wr1, b1)
    return (out, edge_index)
